# baseline (speedup 1.0000x reference)
import functools
import math

import jax
import jax.numpy as jnp
from jax import lax
from jax.experimental import pallas as pl
from jax.experimental.pallas import tpu as pltpu
from jax.experimental.pallas import tpu_sc as plsc

D = 128
B, K, A = 16, 64, 32
N = B * K
C = 6
NW = 32
CHUNK = 128
NBUF = 7
AHEAD = 6
GROUP_COMBOS = (4, 2)
GROUPS = len(GROUP_COMBOS)


def _sc_gather(ent_embed, rel_embed, self_idx, adj_idx, rel_idx):
    sc = self_idx.shape[1]
    ac = adj_idx.shape[1]
    rc = rel_idx.shape[1]
    mesh = plsc.VectorSubcoreMesh(core_axis_name="c", subcore_axis_name="s")

    @functools.partial(
        pl.kernel,
        mesh=mesh,
        out_type=[
            jax.ShapeDtypeStruct((NW * sc * CHUNK, D), jnp.float32),
            jax.ShapeDtypeStruct((NW * ac * CHUNK, D), jnp.float32),
            jax.ShapeDtypeStruct((NW * rc * CHUNK, D), jnp.float32),
        ],
        scratch_types=[
            pltpu.VMEM((sc, CHUNK), jnp.int32),
            pltpu.VMEM((ac, CHUNK), jnp.int32),
            pltpu.VMEM((rc, CHUNK), jnp.int32),
            pltpu.VMEM((NBUF, CHUNK, D), jnp.float32),
            pltpu.SemaphoreType.DMA,
            pltpu.SemaphoreType.DMA,
        ],
    )
    def k(ent_hbm, rel_hbm, selfidx_hbm, adjidx_hbm, relidx_hbm,
          self_out, adj_out, rel_out,
          selfidx_v, adjidx_v, relidx_v, rows_v, gsem, wsem):
        wid = lax.axis_index("s") * 2 + lax.axis_index("c")
        pltpu.sync_copy(selfidx_hbm.at[wid], selfidx_v)
        pltpu.sync_copy(adjidx_hbm.at[wid], adjidx_v)
        pltpu.sync_copy(relidx_hbm.at[wid], relidx_v)

        def seg(idx_v, table_hbm, out_hbm, n_chunks):
            base_w = wid * n_chunks * CHUNK

            def gather(i):
                pltpu.make_async_copy(
                    table_hbm.at[idx_v.at[i]], rows_v.at[i % NBUF],
                    gsem).start()

            def wait_gather():
                pltpu.make_async_copy(
                    table_hbm.at[idx_v.at[0]], rows_v.at[0], gsem).wait()

            def write(i):
                pltpu.make_async_copy(
                    rows_v.at[i % NBUF],
                    out_hbm.at[pl.ds(base_w + i * CHUNK, CHUNK)],
                    wsem).start()

            def wait_write():
                pltpu.make_async_copy(
                    rows_v.at[0], out_hbm.at[pl.ds(base_w, CHUNK)],
                    wsem).wait()

            for j in range(min(AHEAD, n_chunks)):
                gather(j)

            def body(i, carry):
                @pl.when(i >= 2)
                def _():
                    wait_write()

                @pl.when(i + AHEAD < n_chunks)
                def _():
                    gather(i + AHEAD)

                wait_gather()
                write(i)
                return carry

            lax.fori_loop(0, n_chunks, body, 0)
            wait_write()
            if n_chunks >= 2:
                wait_write()

        seg(selfidx_v, ent_hbm, self_out, sc)
        seg(adjidx_v, ent_hbm, adj_out, ac)
        seg(relidx_v, rel_hbm, rel_out, rc)

    return k(ent_embed, rel_embed, self_idx, adj_idx, rel_idx)


TN = 32


def _split_bf16(w):
    wh = w.astype(jnp.bfloat16)
    wl = (w - wh.astype(jnp.float32)).astype(jnp.bfloat16)
    return wh, wl


def _dot3(x, wsplit):
    wh, wl = wsplit
    xh = x.astype(jnp.bfloat16)
    xl = (x - xh.astype(jnp.float32)).astype(jnp.bfloat16)

    def d(a, b):
        return jnp.dot(a, b, preferred_element_type=jnp.float32)

    return d(xh, wh) + (d(xh, wl) + d(xl, wh))


SUB = 16


def _encode_body(self_ref, adj_ref, rel_ref, relidx_ref,
                 wm_ref, bm_ref, wself_ref, out_ref):
    wm = wm_ref[...]
    wm_top = _split_bf16(wm[:D])
    wm_bot = _split_bf16(wm[D:])
    bm = bm_ref[...]
    self_h = jnp.dot(self_ref[...], wself_ref[...],
                     preferred_element_type=jnp.float32,
                     precision=lax.Precision.HIGHEST)
    mask = relidx_ref[...] != 0
    scale = 1.0 / math.sqrt(float(D))
    for j in range(TN // SUB):
        r = slice(j * SUB * A, (j + 1) * SUB * A)
        pre = (_dot3(rel_ref[r, :], wm_top) + _dot3(adj_ref[r, :], wm_bot)
               + bm)
        msg3 = jnp.tanh(pre).reshape(SUB, A, D)
        sh = self_h[j * SUB:(j + 1) * SUB]
        logits = jnp.sum(msg3 * sh[:, None, :], axis=-1) * scale
        logits = jnp.where(mask[j * SUB:(j + 1) * SUB], logits, -1e9)
        lmax = jnp.max(logits, axis=-1, keepdims=True)
        e = jnp.exp(logits - lmax)
        attn = e / jnp.sum(e, axis=-1, keepdims=True)
        agg = jnp.sum(msg3 * attn[:, :, None], axis=1)
        out_ref[j * SUB:(j + 1) * SUB, :] = jnp.maximum(sh + agg, 0.0)


def _tc_encode(self_rows, adj_rows, rel_rows, relidx2d, W_msg, b_msg, W_self):
    n_nodes = relidx2d.shape[0]
    grid = (n_nodes // TN,)
    return pl.pallas_call(
        _encode_body,
        grid=grid,
        in_specs=[
            pl.BlockSpec((TN, D), lambda g: (g, 0)),
            pl.BlockSpec((TN * A, D), lambda g: (g, 0)),
            pl.BlockSpec((TN * A, D), lambda g: (g, 0)),
            pl.BlockSpec((TN, A), lambda g: (g, 0)),
            pl.BlockSpec((2 * D, D), lambda g: (0, 0)),
            pl.BlockSpec((1, D), lambda g: (0, 0)),
            pl.BlockSpec((D, D), lambda g: (0, 0)),
        ],
        out_specs=pl.BlockSpec((TN, D), lambda g: (g, 0)),
        out_shape=jax.ShapeDtypeStruct((n_nodes, D), jnp.float32),
    )(self_rows, adj_rows, rel_rows, relidx2d, W_msg, b_msg, W_self)


def _compare_body(*refs):
    att_refs = refs[:GROUPS]
    w1_ref, b1_ref, w2_ref, b2_ref, out_ref = refs[GROUPS:]
    att = jnp.concatenate([r[...] for r in att_refs], axis=0)
    means = []
    for c in range(C):
        seg = att[c * N:(c + 1) * N, :].reshape(B, K, D)
        means.append(jnp.mean(seg, axis=1))
    support = jnp.concatenate([means[0], means[1]], axis=-1)
    positive = jnp.concatenate([means[2], means[3]], axis=-1)
    negative = jnp.concatenate([means[4], means[5]], axis=-1)
    w1 = w1_ref[...]
    b1 = b1_ref[...]
    w2 = w2_ref[...]
    b2 = b2_ref[...]

    def score(q):
        h = jnp.maximum(
            jnp.dot(jnp.concatenate([support, q], axis=-1), w1,
                    preferred_element_type=jnp.float32,
                    precision=lax.Precision.HIGHEST) + b1, 0.0)
        return jnp.sum(h * w2, axis=-1, keepdims=True) + b2

    pos_s = score(positive)
    neg_s = score(negative)
    out_ref[...] = jnp.concatenate([pos_s, neg_s], axis=1)


def _tc_compare(attn_parts, W1, b1, W2, b2):
    return pl.pallas_call(
        _compare_body,
        out_shape=jax.ShapeDtypeStruct((B, 2), jnp.float32),
    )(*attn_parts, W1, b1, W2, b2)


def _worker_chunks(flat, pad_to_chunks=None):
    total = flat.shape[0]
    per = NW * CHUNK
    n_chunks = -(-total // per)
    if pad_to_chunks is not None:
        n_chunks = pad_to_chunks
    padded = n_chunks * per
    if padded != total:
        flat = jnp.concatenate(
            [flat, jnp.zeros((padded - total,), jnp.int32)])
    return flat.reshape(NW, n_chunks, CHUNK)


def kernel(sup, pos, neg, sup_src_meta, sup_dst_meta, pos_src_meta,
           pos_dst_meta, neg_src_meta, neg_dst_meta, ent_embed, rel_embed,
           W_msg, b_msg, W_self, W1, b1, W2, b2):
    i32 = jnp.int32
    self_idx = jnp.stack([sup[:, 0], sup[:, 1], pos[:, 0], pos[:, 1],
                          neg[:, 0], neg[:, 1]]).astype(i32).reshape(-1)
    adj_idx = jnp.stack([sup_src_meta[:, 1], sup_dst_meta[:, 1],
                         pos_src_meta[:, 1], pos_dst_meta[:, 1],
                         neg_src_meta[:, 1], neg_dst_meta[:, 1]]
                        ).astype(i32).reshape(-1)
    rel_idx = jnp.stack([sup_src_meta[:, 0], sup_dst_meta[:, 0],
                         pos_src_meta[:, 0], pos_dst_meta[:, 0],
                         neg_src_meta[:, 0], neg_dst_meta[:, 0]]
                        ).astype(i32).reshape(-1)

    bm = b_msg.reshape(1, D)
    attn_parts = []
    c_lo = 0
    for cpg in GROUP_COMBOS:
        s_lo, s_hi = c_lo * N, (c_lo + cpg) * N
        a_lo, a_hi = s_lo * A, s_hi * A
        c_lo += cpg
        self_w = _worker_chunks(self_idx[s_lo:s_hi])
        adj_w = _worker_chunks(adj_idx[a_lo:a_hi])
        rel_w = _worker_chunks(rel_idx[a_lo:a_hi])
        self_rows, adj_rows, rel_rows = _sc_gather(
            ent_embed, rel_embed, self_w, adj_w, rel_w)
        attn_parts.append(_tc_encode(
            self_rows, adj_rows, rel_rows,
            rel_idx[a_lo:a_hi].reshape(cpg * N, A), W_msg, bm, W_self))

    scores = _tc_compare(attn_parts, W1, b1.reshape(1, 2 * D),
                         W2.reshape(1, 2 * D), b2.reshape(1, 1))
    return (scores[:, 0], scores[:, 1])


# speedup vs baseline: 1.1690x; 1.1690x over previous
import functools
import math

import jax
import jax.numpy as jnp
from jax import lax
from jax.experimental import pallas as pl
from jax.experimental.pallas import tpu as pltpu
from jax.experimental.pallas import tpu_sc as plsc

D = 128
B, K, A = 16, 64, 32
N = B * K
C = 6
NW = 32
CHUNK = 128
NBUF = 7
AHEAD = 6
GROUP_COMBOS = (4, 2)
GROUPS = len(GROUP_COMBOS)


def _sc_gather(ent_embed, rel_embed, self_idx, adj_idx, rel_idx):
    sc = self_idx.shape[1]
    ac = adj_idx.shape[1]
    rc = rel_idx.shape[1]
    mesh = plsc.VectorSubcoreMesh(core_axis_name="c", subcore_axis_name="s")

    @functools.partial(
        pl.kernel,
        mesh=mesh,
        out_type=[
            jax.ShapeDtypeStruct((NW * sc * CHUNK, D), jnp.float32),
            jax.ShapeDtypeStruct((NW * ac * CHUNK, D), jnp.float32),
            jax.ShapeDtypeStruct((NW * rc * CHUNK, D), jnp.float32),
        ],
        scratch_types=[
            pltpu.VMEM((sc, CHUNK), jnp.int32),
            pltpu.VMEM((ac, CHUNK), jnp.int32),
            pltpu.VMEM((rc, CHUNK), jnp.int32),
            pltpu.VMEM((NBUF, CHUNK, D), jnp.float32),
            pltpu.SemaphoreType.DMA,
            pltpu.SemaphoreType.DMA,
        ],
    )
    def k(ent_hbm, rel_hbm, selfidx_hbm, adjidx_hbm, relidx_hbm,
          self_out, adj_out, rel_out,
          selfidx_v, adjidx_v, relidx_v, rows_v, gsem, wsem):
        wid = lax.axis_index("s") * 2 + lax.axis_index("c")
        pltpu.sync_copy(selfidx_hbm.at[wid], selfidx_v)
        pltpu.sync_copy(adjidx_hbm.at[wid], adjidx_v)
        pltpu.sync_copy(relidx_hbm.at[wid], relidx_v)

        def seg(idx_v, table_hbm, out_hbm, n_chunks):
            base_w = wid * n_chunks * CHUNK

            def gather(i):
                pltpu.make_async_copy(
                    table_hbm.at[idx_v.at[i]], rows_v.at[i % NBUF],
                    gsem).start()

            def wait_gather():
                pltpu.make_async_copy(
                    table_hbm.at[idx_v.at[0]], rows_v.at[0], gsem).wait()

            def write(i):
                pltpu.make_async_copy(
                    rows_v.at[i % NBUF],
                    out_hbm.at[pl.ds(base_w + i * CHUNK, CHUNK)],
                    wsem).start()

            def wait_write():
                pltpu.make_async_copy(
                    rows_v.at[0], out_hbm.at[pl.ds(base_w, CHUNK)],
                    wsem).wait()

            for j in range(min(AHEAD, n_chunks)):
                gather(j)

            def body(i, carry):
                @pl.when(i >= 2)
                def _():
                    wait_write()

                @pl.when(i + AHEAD < n_chunks)
                def _():
                    gather(i + AHEAD)

                wait_gather()
                write(i)
                return carry

            lax.fori_loop(0, n_chunks, body, 0)
            wait_write()
            if n_chunks >= 2:
                wait_write()

        seg(selfidx_v, ent_hbm, self_out, sc)
        seg(adjidx_v, ent_hbm, adj_out, ac)
        seg(relidx_v, rel_hbm, rel_out, rc)

    return k(ent_embed, rel_embed, self_idx, adj_idx, rel_idx)


TN = 256


def _split_bf16(w):
    wh = w.astype(jnp.bfloat16)
    wl = (w - wh.astype(jnp.float32)).astype(jnp.bfloat16)
    return wh, wl


def _dot3(x, wsplit):
    wh, wl = wsplit
    xh = x.astype(jnp.bfloat16)
    xl = (x - xh.astype(jnp.float32)).astype(jnp.bfloat16)

    def d(a, b):
        return jnp.dot(a, b, preferred_element_type=jnp.float32)

    return d(xh, wh) + (d(xh, wl) + d(xl, wh))


SUB = 16


def _encode_body(self_ref, adj_ref, rel_ref, relidx_ref,
                 wm_ref, bm_ref, wself_ref, out_ref):
    wm = wm_ref[...]
    wm_top = _split_bf16(wm[:D])
    wm_bot = _split_bf16(wm[D:])
    bm = bm_ref[...]
    self_h = jnp.dot(self_ref[...], wself_ref[...],
                     preferred_element_type=jnp.float32,
                     precision=lax.Precision.HIGHEST)
    mask = relidx_ref[...] != 0
    scale = 1.0 / math.sqrt(float(D))
    for j in range(TN // SUB):
        r = slice(j * SUB * A, (j + 1) * SUB * A)
        pre = (_dot3(rel_ref[r, :], wm_top) + _dot3(adj_ref[r, :], wm_bot)
               + bm)
        msg3 = jnp.tanh(pre).reshape(SUB, A, D)
        sh = self_h[j * SUB:(j + 1) * SUB]
        logits = jnp.sum(msg3 * sh[:, None, :], axis=-1) * scale
        logits = jnp.where(mask[j * SUB:(j + 1) * SUB], logits, -1e9)
        lmax = jnp.max(logits, axis=-1, keepdims=True)
        e = jnp.exp(logits - lmax)
        attn = e / jnp.sum(e, axis=-1, keepdims=True)
        agg = jnp.sum(msg3 * attn[:, :, None], axis=1)
        out_ref[j * SUB:(j + 1) * SUB, :] = jnp.maximum(sh + agg, 0.0)


def _tc_encode(self_rows, adj_rows, rel_rows, relidx2d, W_msg, b_msg, W_self):
    n_nodes = relidx2d.shape[0]
    grid = (n_nodes // TN,)
    return pl.pallas_call(
        _encode_body,
        grid=grid,
        in_specs=[
            pl.BlockSpec((TN, D), lambda g: (g, 0)),
            pl.BlockSpec((TN * A, D), lambda g: (g, 0)),
            pl.BlockSpec((TN * A, D), lambda g: (g, 0)),
            pl.BlockSpec((TN, A), lambda g: (g, 0)),
            pl.BlockSpec((2 * D, D), lambda g: (0, 0)),
            pl.BlockSpec((1, D), lambda g: (0, 0)),
            pl.BlockSpec((D, D), lambda g: (0, 0)),
        ],
        out_specs=pl.BlockSpec((TN, D), lambda g: (g, 0)),
        out_shape=jax.ShapeDtypeStruct((n_nodes, D), jnp.float32),
    )(self_rows, adj_rows, rel_rows, relidx2d, W_msg, b_msg, W_self)


def _compare_body(*refs):
    att_refs = refs[:GROUPS]
    w1_ref, b1_ref, w2_ref, b2_ref, out_ref = refs[GROUPS:]
    att = jnp.concatenate([r[...] for r in att_refs], axis=0)
    means = []
    for c in range(C):
        seg = att[c * N:(c + 1) * N, :].reshape(B, K, D)
        means.append(jnp.mean(seg, axis=1))
    support = jnp.concatenate([means[0], means[1]], axis=-1)
    positive = jnp.concatenate([means[2], means[3]], axis=-1)
    negative = jnp.concatenate([means[4], means[5]], axis=-1)
    w1 = w1_ref[...]
    b1 = b1_ref[...]
    w2 = w2_ref[...]
    b2 = b2_ref[...]

    def score(q):
        h = jnp.maximum(
            jnp.dot(jnp.concatenate([support, q], axis=-1), w1,
                    preferred_element_type=jnp.float32,
                    precision=lax.Precision.HIGHEST) + b1, 0.0)
        return jnp.sum(h * w2, axis=-1, keepdims=True) + b2

    pos_s = score(positive)
    neg_s = score(negative)
    out_ref[...] = jnp.concatenate([pos_s, neg_s], axis=1)


def _tc_compare(attn_parts, W1, b1, W2, b2):
    return pl.pallas_call(
        _compare_body,
        out_shape=jax.ShapeDtypeStruct((B, 2), jnp.float32),
    )(*attn_parts, W1, b1, W2, b2)


def _worker_chunks(flat, pad_to_chunks=None):
    total = flat.shape[0]
    per = NW * CHUNK
    n_chunks = -(-total // per)
    if pad_to_chunks is not None:
        n_chunks = pad_to_chunks
    padded = n_chunks * per
    if padded != total:
        flat = jnp.concatenate(
            [flat, jnp.zeros((padded - total,), jnp.int32)])
    return flat.reshape(NW, n_chunks, CHUNK)


def kernel(sup, pos, neg, sup_src_meta, sup_dst_meta, pos_src_meta,
           pos_dst_meta, neg_src_meta, neg_dst_meta, ent_embed, rel_embed,
           W_msg, b_msg, W_self, W1, b1, W2, b2):
    i32 = jnp.int32
    self_idx = jnp.stack([sup[:, 0], sup[:, 1], pos[:, 0], pos[:, 1],
                          neg[:, 0], neg[:, 1]]).astype(i32).reshape(-1)
    adj_idx = jnp.stack([sup_src_meta[:, 1], sup_dst_meta[:, 1],
                         pos_src_meta[:, 1], pos_dst_meta[:, 1],
                         neg_src_meta[:, 1], neg_dst_meta[:, 1]]
                        ).astype(i32).reshape(-1)
    rel_idx = jnp.stack([sup_src_meta[:, 0], sup_dst_meta[:, 0],
                         pos_src_meta[:, 0], pos_dst_meta[:, 0],
                         neg_src_meta[:, 0], neg_dst_meta[:, 0]]
                        ).astype(i32).reshape(-1)

    bm = b_msg.reshape(1, D)
    attn_parts = []
    c_lo = 0
    for cpg in GROUP_COMBOS:
        s_lo, s_hi = c_lo * N, (c_lo + cpg) * N
        a_lo, a_hi = s_lo * A, s_hi * A
        c_lo += cpg
        self_w = _worker_chunks(self_idx[s_lo:s_hi])
        adj_w = _worker_chunks(adj_idx[a_lo:a_hi])
        rel_w = _worker_chunks(rel_idx[a_lo:a_hi])
        self_rows, adj_rows, rel_rows = _sc_gather(
            ent_embed, rel_embed, self_w, adj_w, rel_w)
        attn_parts.append(_tc_encode(
            self_rows, adj_rows, rel_rows,
            rel_idx[a_lo:a_hi].reshape(cpg * N, A), W_msg, bm, W_self))

    scores = _tc_compare(attn_parts, W1, b1.reshape(1, 2 * D),
                         W2.reshape(1, 2 * D), b2.reshape(1, 1))
    return (scores[:, 0], scores[:, 1])
